# baseline (speedup 1.0000x reference)
import functools

import jax
import jax.numpy as jnp
from jax import lax
from jax.experimental import pallas as pl
from jax.experimental.pallas import tpu as pltpu
from jax.experimental.pallas import tpu_sc as plsc

OUT_D = 128
N = 4096
B = 128

NW = 32
RPW = N // NW
NCOLV = OUT_D // 16


def _tc_body(roi_ref, out_ref):
    r2 = jax.lax.broadcasted_iota(jnp.uint32, (OUT_D, OUT_D), 0)
    c2 = jax.lax.broadcasted_iota(jnp.uint32, (OUT_D, OUT_D), 1)
    for b in range(B):
        x = roi_ref[b, 0].astype(jnp.uint32)
        y = roi_ref[b, 1].astype(jnp.uint32)
        w = roi_ref[b, 2].astype(jnp.uint32)
        h = roi_ref[b, 3].astype(jnp.uint32)
        inside = ((r2 - y) <= h) & ((c2 - x) <= w)
        out_ref[b] = jnp.where(inside, 1.0, 0.0).astype(jnp.float32)


def _tc_kernel(bbox):
    return pl.pallas_call(
        _tc_body,
        grid=(N // B,),
        in_specs=[
            pl.BlockSpec((B, 4), lambda i: (i, 0), memory_space=pltpu.SMEM)
        ],
        out_specs=pl.BlockSpec((B, OUT_D, OUT_D), lambda i: (i, 0, 0)),
        out_shape=jax.ShapeDtypeStruct((N, OUT_D, OUT_D), jnp.float32),
    )(bbox)


NBUF = 4
CH = N // B


def _tcm_body(roi_ref, out_hbm, *rest):
    bufs = rest[:NBUF]
    sems = rest[NBUF:]
    r2 = jax.lax.broadcasted_iota(jnp.uint32, (OUT_D, OUT_D), 0)
    c2 = jax.lax.broadcasted_iota(jnp.uint32, (OUT_D, OUT_D), 1)

    def fill(chunk, buf):
        def body(b, _):
            base = (chunk * B + b) * 4
            x = roi_ref[base].astype(jnp.uint32)
            y = roi_ref[base + 1].astype(jnp.uint32)
            w = roi_ref[base + 2].astype(jnp.uint32)
            h = roi_ref[base + 3].astype(jnp.uint32)
            inside = ((r2 - y) <= h) & ((c2 - x) <= w)
            buf[b] = jnp.where(inside, 1.0, 0.0).astype(jnp.float32)
            return 0

        lax.fori_loop(0, B, body, 0)

    def start(chunk, buf, sem):
        pltpu.make_async_copy(
            buf, out_hbm.at[pl.ds(chunk * B, B)], sem
        ).start()

    def wait(buf, sem):
        pltpu.make_async_copy(buf, out_hbm.at[pl.ds(0, B)], sem).wait()

    for p in range(NBUF):
        fill(p, bufs[p])
        start(p, bufs[p], sems[p])

    def outer(co, _):
        for p in range(NBUF):
            chunk = co * NBUF + p
            wait(bufs[p], sems[p])
            fill(chunk, bufs[p])
            start(chunk, bufs[p], sems[p])
        return 0

    lax.fori_loop(1, CH // NBUF, outer, 0)
    for p in range(NBUF):
        wait(bufs[p], sems[p])


def _tcm_kernel(bbox):
    return pl.pallas_call(
        _tcm_body,
        in_specs=[pl.BlockSpec(memory_space=pltpu.SMEM)],
        out_specs=pl.BlockSpec(memory_space=pl.ANY),
        out_shape=jax.ShapeDtypeStruct((N, OUT_D, OUT_D), jnp.float32),
        scratch_shapes=(
            [pltpu.VMEM((B, OUT_D, OUT_D), jnp.float32)] * NBUF
            + [pltpu.SemaphoreType.DMA] * NBUF
        ),
    )(bbox)


def _sc_mask_kernel(bbox, nroi=N):
    mesh = plsc.VectorSubcoreMesh(core_axis_name="c", subcore_axis_name="s")
    rpw = nroi // NW

    @functools.partial(
        pl.kernel,
        mesh=mesh,
        out_type=jax.ShapeDtypeStruct((N, OUT_D, OUT_D), jnp.float32),
        scratch_types=[
            pltpu.VMEM((RPW * 4 + 16,), jnp.int32),
            pltpu.VMEM((OUT_D, OUT_D), jnp.float32),
            pltpu.VMEM((OUT_D, OUT_D), jnp.float32),
            pltpu.SemaphoreType.DMA,
            pltpu.SemaphoreType.DMA,
        ],
    )
    def sc_kernel(roi_hbm, out_hbm, roi_v, mask_a, mask_b, sem_a, sem_b):
        cid = lax.axis_index("c")
        sid = lax.axis_index("s")
        wid = sid * 2 + cid
        base = wid * rpw

        pltpu.sync_copy(roi_hbm.at[pl.ds(base * 4, rpw * 4)], roi_v.at[pl.ds(0, rpw * 4)])

        zero16 = jnp.zeros((16,), jnp.float32)

        def zero_row(r, _):
            for j in range(NCOLV):
                mask_a[r, pl.ds(16 * j, 16)] = zero16
                mask_b[r, pl.ds(16 * j, 16)] = zero16
            return 0

        lax.fori_loop(0, OUT_D, zero_row, 0)

        lane16 = lax.broadcasted_iota(jnp.int32, (16,), 0)

        def write_roi(i, buf):
            chunk = roi_v[pl.ds(4 * i, 16)]
            x_s = chunk[0]
            y_s = chunk[1]
            xw_s = x_s + chunk[2]
            h_s = chunk[3]
            cind = []
            for j in range(NCOLV):
                cj = lane16 + (16 * j)
                cind.append(
                    jnp.where((cj >= x_s) & (cj <= xw_s), 1.0, 0.0).astype(
                        jnp.float32
                    )
                )

            def row(r, _):
                for j in range(NCOLV):
                    buf[r, pl.ds(16 * j, 16)] = cind[j]
                return 0

            lax.fori_loop(y_s, y_s + h_s + 1, row, 0)
            return y_s, h_s

        def clear_rows(buf, y_s, h_s):
            def row(r, _):
                for j in range(NCOLV):
                    buf[r, pl.ds(16 * j, 16)] = zero16
                return 0

            lax.fori_loop(y_s, y_s + h_s + 1, row, 0)

        ya, ha = write_roi(0, mask_a)
        pltpu.async_copy(mask_a, out_hbm.at[base], sem_a)
        yb, hb = write_roi(1, mask_b)
        pltpu.async_copy(mask_b, out_hbm.at[base + 1], sem_b)

        def step(k, carry):
            ya, ha, yb, hb = carry
            i0 = 2 * k
            i1 = 2 * k + 1
            pltpu.make_async_copy(mask_a, out_hbm.at[base], sem_a).wait()
            clear_rows(mask_a, ya, ha)
            ya, ha = write_roi(i0, mask_a)
            pltpu.async_copy(mask_a, out_hbm.at[base + i0], sem_a)
            pltpu.make_async_copy(mask_b, out_hbm.at[base], sem_b).wait()
            clear_rows(mask_b, yb, hb)
            yb, hb = write_roi(i1, mask_b)
            pltpu.async_copy(mask_b, out_hbm.at[base + i1], sem_b)
            return ya, ha, yb, hb

        lax.fori_loop(1, rpw // 2, step, (ya, ha, yb, hb))

        pltpu.make_async_copy(mask_a, out_hbm.at[base], sem_a).wait()
        pltpu.make_async_copy(mask_b, out_hbm.at[base], sem_b).wait()

    return sc_kernel(bbox)


K_SC = 1024


def _tc_tail_body(roi_ref, alias_ref, out_ref):
    del alias_ref
    _tc_body(roi_ref, out_ref)


def _hybrid_kernel(bbox2d, bbox_flat):
    sc_out = _sc_mask_kernel(bbox_flat, K_SC)
    kb = K_SC // B
    return pl.pallas_call(
        _tc_tail_body,
        grid=((N - K_SC) // B,),
        in_specs=[
            pl.BlockSpec(
                (B, 4), lambda i, kb=kb: (i + kb, 0), memory_space=pltpu.SMEM
            ),
            pl.BlockSpec(memory_space=pl.ANY),
        ],
        out_specs=pl.BlockSpec(
            (B, OUT_D, OUT_D), lambda i, kb=kb: (i + kb, 0, 0)
        ),
        out_shape=jax.ShapeDtypeStruct((N, OUT_D, OUT_D), jnp.float32),
        input_output_aliases={1: 0},
    )(bbox2d, sc_out)


def kernel(output_roi):
    bbox = output_roi[:, 1:5].astype(jnp.int32)
    return _hybrid_kernel(bbox, bbox.reshape(-1))


# speedup vs baseline: 1.2489x; 1.2489x over previous
import functools

import jax
import jax.numpy as jnp
from jax import lax
from jax.experimental import pallas as pl
from jax.experimental.pallas import tpu as pltpu
from jax.experimental.pallas import tpu_sc as plsc

OUT_D = 128
N = 4096
B = 128

NW = 32
RPW = N // NW
NCOLV = OUT_D // 16


def _tc_body(roi_ref, out_ref):
    r2 = jax.lax.broadcasted_iota(jnp.uint32, (OUT_D, OUT_D), 0)
    c2 = jax.lax.broadcasted_iota(jnp.uint32, (OUT_D, OUT_D), 1)
    for b in range(B):
        x = roi_ref[b, 0].astype(jnp.uint32)
        y = roi_ref[b, 1].astype(jnp.uint32)
        w = roi_ref[b, 2].astype(jnp.uint32)
        h = roi_ref[b, 3].astype(jnp.uint32)
        inside = ((r2 - y) <= h) & ((c2 - x) <= w)
        out_ref[b] = jnp.where(inside, 1.0, 0.0).astype(jnp.float32)


def _tc_kernel(bbox):
    return pl.pallas_call(
        _tc_body,
        grid=(N // B,),
        in_specs=[
            pl.BlockSpec((B, 4), lambda i: (i, 0), memory_space=pltpu.SMEM)
        ],
        out_specs=pl.BlockSpec((B, OUT_D, OUT_D), lambda i: (i, 0, 0)),
        out_shape=jax.ShapeDtypeStruct((N, OUT_D, OUT_D), jnp.float32),
    )(bbox)


NBUF = 4
CH = N // B


def _tcm_body(roi_ref, out_hbm, *rest):
    bufs = rest[:NBUF]
    sems = rest[NBUF:]
    r2 = jax.lax.broadcasted_iota(jnp.uint32, (OUT_D, OUT_D), 0)
    c2 = jax.lax.broadcasted_iota(jnp.uint32, (OUT_D, OUT_D), 1)

    def fill(chunk, buf):
        def body(b, _):
            base = (chunk * B + b) * 4
            x = roi_ref[base].astype(jnp.uint32)
            y = roi_ref[base + 1].astype(jnp.uint32)
            w = roi_ref[base + 2].astype(jnp.uint32)
            h = roi_ref[base + 3].astype(jnp.uint32)
            inside = ((r2 - y) <= h) & ((c2 - x) <= w)
            buf[b] = jnp.where(inside, 1.0, 0.0).astype(jnp.float32)
            return 0

        lax.fori_loop(0, B, body, 0)

    def start(chunk, buf, sem):
        pltpu.make_async_copy(
            buf, out_hbm.at[pl.ds(chunk * B, B)], sem
        ).start()

    def wait(buf, sem):
        pltpu.make_async_copy(buf, out_hbm.at[pl.ds(0, B)], sem).wait()

    for p in range(NBUF):
        fill(p, bufs[p])
        start(p, bufs[p], sems[p])

    def outer(co, _):
        for p in range(NBUF):
            chunk = co * NBUF + p
            wait(bufs[p], sems[p])
            fill(chunk, bufs[p])
            start(chunk, bufs[p], sems[p])
        return 0

    lax.fori_loop(1, CH // NBUF, outer, 0)
    for p in range(NBUF):
        wait(bufs[p], sems[p])


def _tcm_kernel(bbox):
    return pl.pallas_call(
        _tcm_body,
        in_specs=[pl.BlockSpec(memory_space=pltpu.SMEM)],
        out_specs=pl.BlockSpec(memory_space=pl.ANY),
        out_shape=jax.ShapeDtypeStruct((N, OUT_D, OUT_D), jnp.float32),
        scratch_shapes=(
            [pltpu.VMEM((B, OUT_D, OUT_D), jnp.float32)] * NBUF
            + [pltpu.SemaphoreType.DMA] * NBUF
        ),
    )(bbox)


def _sc_mask_kernel(bbox, nroi=N):
    mesh = plsc.VectorSubcoreMesh(core_axis_name="c", subcore_axis_name="s")
    rpw = nroi // NW

    @functools.partial(
        pl.kernel,
        mesh=mesh,
        out_type=jax.ShapeDtypeStruct((N, OUT_D, OUT_D), jnp.float32),
        scratch_types=[
            pltpu.VMEM((RPW * 4 + 16,), jnp.int32),
            pltpu.VMEM((OUT_D, OUT_D), jnp.float32),
            pltpu.VMEM((OUT_D, OUT_D), jnp.float32),
            pltpu.SemaphoreType.DMA,
            pltpu.SemaphoreType.DMA,
        ],
    )
    def sc_kernel(roi_hbm, out_hbm, roi_v, mask_a, mask_b, sem_a, sem_b):
        cid = lax.axis_index("c")
        sid = lax.axis_index("s")
        wid = sid * 2 + cid
        base = wid * rpw

        pltpu.sync_copy(roi_hbm.at[pl.ds(base * 4, rpw * 4)], roi_v.at[pl.ds(0, rpw * 4)])

        zero16 = jnp.zeros((16,), jnp.float32)

        def zero_row(r, _):
            for j in range(NCOLV):
                mask_a[r, pl.ds(16 * j, 16)] = zero16
                mask_b[r, pl.ds(16 * j, 16)] = zero16
            return 0

        lax.fori_loop(0, OUT_D, zero_row, 0)

        lane16 = lax.broadcasted_iota(jnp.int32, (16,), 0)

        def write_roi(i, buf):
            chunk = roi_v[pl.ds(4 * i, 16)]
            x_s = chunk[0]
            y_s = chunk[1]
            xw_s = x_s + chunk[2]
            h_s = chunk[3]
            cind = []
            for j in range(NCOLV):
                cj = lane16 + (16 * j)
                cind.append(
                    jnp.where((cj >= x_s) & (cj <= xw_s), 1.0, 0.0).astype(
                        jnp.float32
                    )
                )

            def row(r, _):
                for j in range(NCOLV):
                    buf[r, pl.ds(16 * j, 16)] = cind[j]
                return 0

            lax.fori_loop(y_s, y_s + h_s + 1, row, 0)
            return y_s, h_s

        def clear_rows(buf, y_s, h_s):
            def row(r, _):
                for j in range(NCOLV):
                    buf[r, pl.ds(16 * j, 16)] = zero16
                return 0

            lax.fori_loop(y_s, y_s + h_s + 1, row, 0)

        ya, ha = write_roi(0, mask_a)
        pltpu.async_copy(mask_a, out_hbm.at[base], sem_a)
        yb, hb = write_roi(1, mask_b)
        pltpu.async_copy(mask_b, out_hbm.at[base + 1], sem_b)

        def step(k, carry):
            ya, ha, yb, hb = carry
            i0 = 2 * k
            i1 = 2 * k + 1
            pltpu.make_async_copy(mask_a, out_hbm.at[base], sem_a).wait()
            clear_rows(mask_a, ya, ha)
            ya, ha = write_roi(i0, mask_a)
            pltpu.async_copy(mask_a, out_hbm.at[base + i0], sem_a)
            pltpu.make_async_copy(mask_b, out_hbm.at[base], sem_b).wait()
            clear_rows(mask_b, yb, hb)
            yb, hb = write_roi(i1, mask_b)
            pltpu.async_copy(mask_b, out_hbm.at[base + i1], sem_b)
            return ya, ha, yb, hb

        lax.fori_loop(1, rpw // 2, step, (ya, ha, yb, hb))

        pltpu.make_async_copy(mask_a, out_hbm.at[base], sem_a).wait()
        pltpu.make_async_copy(mask_b, out_hbm.at[base], sem_b).wait()

    return sc_kernel(bbox)


K_SC = 1024


def _tc_tail_body(roi_ref, alias_ref, out_ref):
    del alias_ref
    _tc_body(roi_ref, out_ref)


def _hybrid_kernel(bbox2d, bbox_flat):
    sc_out = _sc_mask_kernel(bbox_flat, K_SC)
    kb = K_SC // B
    return pl.pallas_call(
        _tc_tail_body,
        grid=((N - K_SC) // B,),
        in_specs=[
            pl.BlockSpec(
                (B, 4), lambda i, kb=kb: (i + kb, 0), memory_space=pltpu.SMEM
            ),
            pl.BlockSpec(memory_space=pl.ANY),
        ],
        out_specs=pl.BlockSpec(
            (B, OUT_D, OUT_D), lambda i, kb=kb: (i + kb, 0, 0)
        ),
        out_shape=jax.ShapeDtypeStruct((N, OUT_D, OUT_D), jnp.float32),
        input_output_aliases={1: 0},
    )(bbox2d, sc_out)


def kernel(output_roi):
    bbox = output_roi[:, 1:5].astype(jnp.int32)
    return _tc_kernel(bbox)
